# baseline (speedup 1.0000x reference)
import jax
import jax.numpy as jnp
from jax.experimental import pallas as pl


def _copy_body(x_ref, o_ref):
    o_ref[...] = x_ref[...]


def kernel(x, batch, W_gate, b_gate):
    G = 1024
    gate = x @ W_gate.T + b_gate
    seg_max = jax.ops.segment_max(gate, batch, num_segments=G)
    seg_max = jnp.where(jnp.isfinite(seg_max), seg_max, 0.0)
    e = jnp.exp(gate - seg_max[batch])
    denom = jax.ops.segment_sum(e, batch, num_segments=G)
    attn = e / (denom[batch] + 1e-16)
    out = jax.ops.segment_sum(attn * x, batch, num_segments=G)
    return pl.pallas_call(
        _copy_body,
        out_shape=jax.ShapeDtypeStruct(out.shape, out.dtype),
    )(out)


# speedup vs baseline: 7.7356x; 7.7356x over previous
import functools

import jax
import jax.numpy as jnp
from jax import lax
from jax.experimental import pallas as pl
from jax.experimental.pallas import tpu as pltpu
from jax.experimental.pallas import tpu_sc as plsc

N = 100000
D = 128
G = 1024
NC = 2
NS = 16
NW = NC * NS
R = 128
NCHUNK = (N + R - 1) // R
FULL_TRIPS = NCHUNK // NW
EXTRA_W = NCHUNK - FULL_TRIPS * NW
SLOTS = 16
ACC_ROWS = G + 128


def _sc_body(x_hbm, batch_hbm, w_hbm, b_hbm, outf_hbm, outd_hbm,
             xbuf, bbat, wbuf, bbuf, stage, stage_d, istate, obuf,
             acc_sh, den_sh, xsem):
    cid = lax.axis_index("c")
    sid = lax.axis_index("s")
    wid = cid * NS + sid

    zeros16 = jnp.zeros((16,), jnp.float32)
    rows_per_tec = ACC_ROWS // NS

    def _zrow(r, _):
        for j in range(8):
            obuf[r, pl.ds(j * 16, 16)] = zeros16
        return 0

    lax.fori_loop(0, rows_per_tec, _zrow, 0)
    pltpu.sync_copy(obuf.at[pl.ds(0, rows_per_tec)],
                    acc_sh.at[pl.ds(sid * rows_per_tec, rows_per_tec)])
    pltpu.sync_copy(obuf.at[pl.ds(0, rows_per_tec)],
                    den_sh.at[pl.ds(sid * rows_per_tec, rows_per_tec)])
    for r in range(SLOTS + 8):
        for j in range(8):
            stage_d[r, pl.ds(j * 16, 16)] = zeros16
    plsc.subcore_barrier()

    pltpu.sync_copy(w_hbm, wbuf)
    pltpu.sync_copy(b_hbm, bbuf)
    wv = [wbuf[pl.ds(j * 16, 16)] for j in range(8)]
    bv = bbuf[...]
    lane = lax.broadcasted_iota(jnp.int32, (16,), 0)
    bfly = [lane ^ s for s in (8, 4, 2, 1)]
    onehot0 = jnp.where(lane == 0, 1.0, 0.0)
    dump_idx = lane * 0 + G

    def _flush():
        pltpu.sync_copy(stage.at[pl.ds(0, SLOTS)], acc_sh.at[istate], add=True)
        pltpu.sync_copy(stage_d.at[pl.ds(0, SLOTS)], den_sh.at[istate], add=True)

    def _stage_row(k, a, dvec):
        for j in range(8):
            stage[k, pl.ds(j * 16, 16)] = a[j]
        stage_d[k, pl.ds(0, 16)] = dvec

    def _row_fast(r, b, a, dvec):
        xv = [xbuf[b + r, pl.ds(j * 16, 16)] for j in range(8)]
        gv = xv[0] * wv[0]
        for j in range(1, 8):
            gv = gv + xv[j] * wv[j]
        for ix in bfly:
            gv = gv + gv.at[ix].get(mode="promise_in_bounds")
        ev = jnp.exp(gv + bv)
        a = tuple(a[j] + ev * xv[j] for j in range(8))
        dvec = dvec + ev * onehot0
        return a, dvec

    SHTL = SLOTS

    def _chunk_rows(ci, b, loff):
        cur_seg0 = bbat[pl.ds(loff, 16)][0]

        def _group(grp, gcarry):
            cur_seg, k, a, dvec = gcarry
            bat16 = bbat[pl.ds(grp * 16, 16)]
            s_idx = istate[...]
            for i in range(16):
                r = grp * 16 + i
                seg = bat16[i]
                new_i = jnp.where(seg != cur_seg, 1, 0)
                newv = lane * 0 + new_i
                cur_seg_v = lane * 0 + cur_seg
                slot = 15 if i == 0 else i - 1
                selv = newv * jnp.where(lane == slot, 1, 0)
                s_idx = jnp.where(selv > 0, cur_seg_v, s_idx)
                k = k + new_i
                keep = jnp.where(new_i == 1, 0.0, 1.0)

                xv = [xbuf[b + r, pl.ds(j * 16, 16)] for j in range(8)]
                gv = xv[0] * wv[0]
                for j in range(1, 8):
                    gv = gv + xv[j] * wv[j]
                for ix in bfly:
                    gv = gv + gv.at[ix].get(mode="promise_in_bounds")
                ev = jnp.exp(gv + bv)
                a = tuple(a[j] * keep + ev * xv[j] for j in range(8))
                dvec = dvec * keep + ev * onehot0
                if i < 15:
                    _stage_row(i, a, dvec)
                cur_seg = seg
            istate[...] = s_idx

            @pl.when(k > 0)
            def _():
                _flush()

            istate[...] = dump_idx
            _stage_row(15, a, dvec)
            return cur_seg, k * 0, a, dvec

        acc0 = tuple(jnp.zeros((16,), jnp.float32) for _ in range(8))
        gcarry = (cur_seg0, jnp.int32(0), acc0, zeros16)
        cur_seg, k, a, dvec = lax.fori_loop(
            loff // 16, R // 16, _group, gcarry)

        _stage_row(0, a, dvec)
        cur_seg_v = lane * 0 + cur_seg
        istate[...] = jnp.where(lane == 0, cur_seg_v, dump_idx)
        _flush()
        istate[...] = dump_idx

    def _ci_row0(c):
        ci = jnp.minimum(wid + c * NW, NCHUNK - 1)
        row0 = jnp.minimum(ci * R, N - R)
        return ci, row0

    def _chunk(c, _):
        ci, row0 = _ci_row0(c)
        loff = (wid + c * NW) * R - row0
        boff = (c % 2) * R
        pltpu.make_async_copy(x_hbm.at[pl.ds(row0, R)],
                              xbuf.at[pl.ds(boff, R)], xsem).wait()
        _, row0n = _ci_row0(c + 1)
        pltpu.async_copy(x_hbm.at[pl.ds(row0n, R)],
                         xbuf.at[pl.ds(R - boff, R)], xsem)
        pltpu.sync_copy(batch_hbm.at[pl.ds(row0, R)], bbat)
        _chunk_rows(ci, boff, loff)
        return 0

    istate[...] = dump_idx
    ntrips = FULL_TRIPS + jnp.where(wid < EXTRA_W, 1, 0)
    _, row0p = _ci_row0(0)
    pltpu.async_copy(x_hbm.at[pl.ds(row0p, R)], xbuf.at[pl.ds(0, R)], xsem)
    lax.fori_loop(0, ntrips, _chunk, 0)
    _, row0d = _ci_row0(ntrips)
    pltpu.make_async_copy(x_hbm.at[pl.ds(row0d, R)],
                          xbuf.at[pl.ds((ntrips % 2) * R, R)], xsem).wait()

    plsc.subcore_barrier()
    out_rows = G // NS
    pltpu.sync_copy(acc_sh.at[pl.ds(sid * out_rows, out_rows)],
                    obuf.at[pl.ds(0, out_rows)])
    pltpu.sync_copy(obuf.at[pl.ds(0, out_rows)],
                    outf_hbm.at[cid, pl.ds(sid * out_rows, out_rows)])
    pltpu.sync_copy(den_sh.at[pl.ds(sid * out_rows, out_rows)],
                    obuf.at[pl.ds(0, out_rows)])
    pltpu.sync_copy(obuf.at[pl.ds(0, out_rows)],
                    outd_hbm.at[cid, pl.ds(sid * out_rows, out_rows)])


@functools.partial(
    pl.kernel,
    out_type=(jax.ShapeDtypeStruct((NC, G, D), jnp.float32),
              jax.ShapeDtypeStruct((NC, G, D), jnp.float32)),
    mesh=plsc.VectorSubcoreMesh(core_axis_name="c", subcore_axis_name="s"),
    scratch_types=[
        pltpu.VMEM((2 * R, D), jnp.float32),
        pltpu.VMEM((R,), jnp.int32),
        pltpu.VMEM((D,), jnp.float32),
        pltpu.VMEM((16,), jnp.float32),
        pltpu.VMEM((SLOTS + 8, D), jnp.float32),
        pltpu.VMEM((SLOTS + 8, D), jnp.float32),
        pltpu.VMEM((SLOTS,), jnp.int32),
        pltpu.VMEM((ACC_ROWS // NS, D), jnp.float32),
        pltpu.VMEM_SHARED((ACC_ROWS, D), jnp.float32),
        pltpu.VMEM_SHARED((ACC_ROWS, D), jnp.float32),
        pltpu.SemaphoreType.DMA,
    ],
)
def _sc_pool(x_hbm, batch_hbm, w_hbm, b_hbm, outf_hbm, outd_hbm, *scratch):
    _sc_body(x_hbm, batch_hbm, w_hbm, b_hbm, outf_hbm, outd_hbm, *scratch)


def _merge_body(f0_ref, f1_ref, d0_ref, d1_ref, o_ref):
    s = f0_ref[...] + f1_ref[...]
    den = d0_ref[...] + d1_ref[...]
    d = jnp.sum(den, axis=1, keepdims=True)
    o_ref[...] = s / (d + 1e-16)


def kernel(x, batch, W_gate, b_gate):
    w128 = W_gate.reshape(D)
    b16 = jnp.broadcast_to(b_gate, (16,)).astype(jnp.float32)
    pf, pd = _sc_pool(x, batch, w128, b16)
    out = pl.pallas_call(
        _merge_body,
        out_shape=jax.ShapeDtypeStruct((G, D), jnp.float32),
    )(pf[0], pf[1], pd[0], pd[1])
    return out


# speedup vs baseline: 11.4119x; 1.4753x over previous
import functools

import jax
import jax.numpy as jnp
from jax import lax
from jax.experimental import pallas as pl
from jax.experimental.pallas import tpu as pltpu
from jax.experimental.pallas import tpu_sc as plsc

N = 100000
D = 128
G = 1024
NC = 2
NS = 16
NW = NC * NS
R = 128
NCHUNK = (N + R - 1) // R
FULL_TRIPS = NCHUNK // NW
EXTRA_W = NCHUNK - FULL_TRIPS * NW
SLOTS = 16
ACC_ROWS = G + 128


def _sc_body(x_hbm, batch_hbm, w_hbm, b_hbm, outf_hbm, outd_hbm,
             xbuf, bbat, wbuf, bbuf, stage, stage_d, istate, obuf,
             acc_sh, den_sh, xsem):
    cid = lax.axis_index("c")
    sid = lax.axis_index("s")
    wid = cid * NS + sid

    zeros16 = jnp.zeros((16,), jnp.float32)
    rows_per_tec = ACC_ROWS // NS

    def _zrow(r, _):
        for j in range(8):
            obuf[r, pl.ds(j * 16, 16)] = zeros16
        return 0

    lax.fori_loop(0, rows_per_tec, _zrow, 0)
    pltpu.sync_copy(obuf.at[pl.ds(0, rows_per_tec)],
                    acc_sh.at[pl.ds(sid * rows_per_tec, rows_per_tec)])
    pltpu.sync_copy(obuf.at[pl.ds(0, rows_per_tec)],
                    den_sh.at[pl.ds(sid * rows_per_tec, rows_per_tec)])
    for r in range(SLOTS + 8):
        for j in range(8):
            stage_d[r, pl.ds(j * 16, 16)] = zeros16
    plsc.subcore_barrier()

    pltpu.sync_copy(w_hbm, wbuf)
    pltpu.sync_copy(b_hbm, bbuf)
    wv = [wbuf[pl.ds(j * 16, 16)] for j in range(8)]
    bv = bbuf[...]
    lane = lax.broadcasted_iota(jnp.int32, (16,), 0)
    bfly = [lane ^ s for s in (8, 4, 2, 1)]
    onehot0 = jnp.where(lane == 0, 1.0, 0.0)
    dump_idx = lane * 0 + G

    def _flush():
        pltpu.sync_copy(stage.at[pl.ds(0, SLOTS)], acc_sh.at[istate], add=True)
        pltpu.sync_copy(stage_d.at[pl.ds(0, SLOTS)], den_sh.at[istate], add=True)

    def _stage_row(k, a, dvec):
        for j in range(8):
            stage[k, pl.ds(j * 16, 16)] = a[j]
        stage_d[k, pl.ds(0, 16)] = dvec

    def _row_fast(r, b, a, dvec):
        xv = [xbuf[b + r, pl.ds(j * 16, 16)] for j in range(8)]
        gv = xv[0] * wv[0]
        for j in range(1, 8):
            gv = gv + xv[j] * wv[j]
        for ix in bfly:
            gv = gv + gv.at[ix].get(mode="promise_in_bounds")
        ev = jnp.exp(gv + bv)
        a = tuple(a[j] + ev * xv[j] for j in range(8))
        dvec = dvec + ev * onehot0
        return a, dvec

    SHTL = SLOTS

    def _chunk_rows(ci, b, loff):
        cur_seg0 = bbat[pl.ds(loff, 16)][0]

        def _gate(r):
            xv = [xbuf[b + r, pl.ds(j * 16, 16)] for j in range(8)]
            p = [xv[j] * wv[j] for j in range(8)]
            q = [p[0] + p[1], p[2] + p[3], p[4] + p[5], p[6] + p[7]]
            gv = (q[0] + q[1]) + (q[2] + q[3])
            for ix in bfly:
                gv = gv + gv.at[ix].get(mode="promise_in_bounds")
            return xv, jnp.exp(gv + bv)

        def _group(grp, gcarry):
            cur_seg, k, a, dvec = gcarry
            bat16 = bbat[pl.ds(grp * 16, 16)]
            s_idx = istate[...]
            xv_c, ev_c = _gate(grp * 16)
            for i in range(16):
                if i < 15:
                    xv_n, ev_n = _gate(grp * 16 + i + 1)
                seg = bat16[i]
                new_i = jnp.where(seg != cur_seg, 1, 0)
                newv = lane * 0 + new_i
                cur_seg_v = lane * 0 + cur_seg
                slot = 15 if i == 0 else i - 1
                selv = newv * jnp.where(lane == slot, 1, 0)
                s_idx = jnp.where(selv > 0, cur_seg_v, s_idx)
                k = k + new_i
                keep = jnp.where(new_i == 1, 0.0, 1.0)
                a = tuple(a[j] * keep + ev_c * xv_c[j] for j in range(8))
                dvec = dvec * keep + ev_c * onehot0
                if i < 15:
                    _stage_row(i, a, dvec)
                    xv_c, ev_c = xv_n, ev_n
                cur_seg = seg
            istate[...] = s_idx

            @pl.when(k > 0)
            def _():
                _flush()

            istate[...] = dump_idx
            _stage_row(15, a, dvec)
            return cur_seg, k * 0, a, dvec

        acc0 = tuple(jnp.zeros((16,), jnp.float32) for _ in range(8))
        gcarry = (cur_seg0, jnp.int32(0), acc0, zeros16)
        cur_seg, k, a, dvec = lax.fori_loop(
            loff // 16, R // 16, _group, gcarry)

        _stage_row(0, a, dvec)
        cur_seg_v = lane * 0 + cur_seg
        istate[...] = jnp.where(lane == 0, cur_seg_v, dump_idx)
        _flush()
        istate[...] = dump_idx

    def _ci_row0(c):
        ci = jnp.minimum(wid + c * NW, NCHUNK - 1)
        row0 = jnp.minimum(ci * R, N - R)
        return ci, row0

    def _chunk(c, _):
        ci, row0 = _ci_row0(c)
        loff = (wid + c * NW) * R - row0
        boff = (c % 2) * R
        pltpu.make_async_copy(x_hbm.at[pl.ds(row0, R)],
                              xbuf.at[pl.ds(boff, R)], xsem).wait()
        _, row0n = _ci_row0(c + 1)
        pltpu.async_copy(x_hbm.at[pl.ds(row0n, R)],
                         xbuf.at[pl.ds(R - boff, R)], xsem)
        pltpu.sync_copy(batch_hbm.at[pl.ds(row0, R)], bbat)
        _chunk_rows(ci, boff, loff)
        return 0

    istate[...] = dump_idx
    ntrips = FULL_TRIPS + jnp.where(wid < EXTRA_W, 1, 0)
    _, row0p = _ci_row0(0)
    pltpu.async_copy(x_hbm.at[pl.ds(row0p, R)], xbuf.at[pl.ds(0, R)], xsem)
    lax.fori_loop(0, ntrips, _chunk, 0)
    _, row0d = _ci_row0(ntrips)
    pltpu.make_async_copy(x_hbm.at[pl.ds(row0d, R)],
                          xbuf.at[pl.ds((ntrips % 2) * R, R)], xsem).wait()

    plsc.subcore_barrier()
    out_rows = G // NS
    pltpu.sync_copy(acc_sh.at[pl.ds(sid * out_rows, out_rows)],
                    obuf.at[pl.ds(0, out_rows)])
    pltpu.sync_copy(obuf.at[pl.ds(0, out_rows)],
                    outf_hbm.at[cid, pl.ds(sid * out_rows, out_rows)])
    pltpu.sync_copy(den_sh.at[pl.ds(sid * out_rows, out_rows)],
                    obuf.at[pl.ds(0, out_rows)])
    pltpu.sync_copy(obuf.at[pl.ds(0, out_rows)],
                    outd_hbm.at[cid, pl.ds(sid * out_rows, out_rows)])


@functools.partial(
    pl.kernel,
    out_type=(jax.ShapeDtypeStruct((NC, G, D), jnp.float32),
              jax.ShapeDtypeStruct((NC, G, D), jnp.float32)),
    mesh=plsc.VectorSubcoreMesh(core_axis_name="c", subcore_axis_name="s"),
    scratch_types=[
        pltpu.VMEM((2 * R, D), jnp.float32),
        pltpu.VMEM((R,), jnp.int32),
        pltpu.VMEM((D,), jnp.float32),
        pltpu.VMEM((16,), jnp.float32),
        pltpu.VMEM((SLOTS + 8, D), jnp.float32),
        pltpu.VMEM((SLOTS + 8, D), jnp.float32),
        pltpu.VMEM((SLOTS,), jnp.int32),
        pltpu.VMEM((ACC_ROWS // NS, D), jnp.float32),
        pltpu.VMEM_SHARED((ACC_ROWS, D), jnp.float32),
        pltpu.VMEM_SHARED((ACC_ROWS, D), jnp.float32),
        pltpu.SemaphoreType.DMA,
    ],
)
def _sc_pool(x_hbm, batch_hbm, w_hbm, b_hbm, outf_hbm, outd_hbm, *scratch):
    _sc_body(x_hbm, batch_hbm, w_hbm, b_hbm, outf_hbm, outd_hbm, *scratch)


def _merge_body(f0_ref, f1_ref, d0_ref, d1_ref, o_ref):
    s = f0_ref[...] + f1_ref[...]
    den = d0_ref[...] + d1_ref[...]
    d = jnp.sum(den, axis=1, keepdims=True)
    o_ref[...] = s / (d + 1e-16)


def kernel(x, batch, W_gate, b_gate):
    w128 = W_gate.reshape(D)
    b16 = jnp.broadcast_to(b_gate, (16,)).astype(jnp.float32)
    pf, pd = _sc_pool(x, batch, w128, b16)
    out = pl.pallas_call(
        _merge_body,
        out_shape=jax.ShapeDtypeStruct((G, D), jnp.float32),
    )(pf[0], pf[1], pd[0], pd[1])
    return out


# speedup vs baseline: 11.8544x; 1.0388x over previous
import functools

import jax
import jax.numpy as jnp
from jax import lax
from jax.experimental import pallas as pl
from jax.experimental.pallas import tpu as pltpu
from jax.experimental.pallas import tpu_sc as plsc

N = 100000
D = 128
G = 1024
NC = 2
NS = 16
NW = NC * NS
R = 256
NCHUNK = (N + R - 1) // R
FULL_TRIPS = NCHUNK // NW
EXTRA_W = NCHUNK - FULL_TRIPS * NW
SLOTS = 16
ACC_ROWS = G + 128


def _sc_body(x_hbm, batch_hbm, w_hbm, b_hbm, outf_hbm, outd_hbm,
             xbuf, bbat, wbuf, bbuf, stage, stage_d, istate, obuf,
             acc_sh, den_sh, xsem):
    cid = lax.axis_index("c")
    sid = lax.axis_index("s")
    wid = cid * NS + sid

    zeros16 = jnp.zeros((16,), jnp.float32)
    rows_per_tec = ACC_ROWS // NS

    def _zrow(r, _):
        for j in range(8):
            obuf[r, pl.ds(j * 16, 16)] = zeros16
        return 0

    lax.fori_loop(0, rows_per_tec, _zrow, 0)
    pltpu.sync_copy(obuf.at[pl.ds(0, rows_per_tec)],
                    acc_sh.at[pl.ds(sid * rows_per_tec, rows_per_tec)])
    pltpu.sync_copy(obuf.at[pl.ds(0, rows_per_tec)],
                    den_sh.at[pl.ds(sid * rows_per_tec, rows_per_tec)])
    for r in range(SLOTS + 8):
        for j in range(8):
            stage_d[r, pl.ds(j * 16, 16)] = zeros16
    plsc.subcore_barrier()

    pltpu.sync_copy(w_hbm, wbuf)
    pltpu.sync_copy(b_hbm, bbuf)
    wv = [wbuf[pl.ds(j * 16, 16)] for j in range(8)]
    bv = bbuf[...]
    lane = lax.broadcasted_iota(jnp.int32, (16,), 0)
    bfly = [lane ^ s for s in (8, 4, 2, 1)]
    onehot0 = jnp.where(lane == 0, 1.0, 0.0)
    dump_idx = lane * 0 + G

    def _flush():
        pltpu.sync_copy(stage.at[pl.ds(0, SLOTS)], acc_sh.at[istate], add=True)
        pltpu.sync_copy(stage_d.at[pl.ds(0, SLOTS)], den_sh.at[istate], add=True)

    def _stage_row(k, a, dvec):
        for j in range(8):
            stage[k, pl.ds(j * 16, 16)] = a[j]
        stage_d[k, pl.ds(0, 16)] = dvec

    def _row_fast(r, b, a, dvec):
        xv = [xbuf[b + r, pl.ds(j * 16, 16)] for j in range(8)]
        gv = xv[0] * wv[0]
        for j in range(1, 8):
            gv = gv + xv[j] * wv[j]
        for ix in bfly:
            gv = gv + gv.at[ix].get(mode="promise_in_bounds")
        ev = jnp.exp(gv + bv)
        a = tuple(a[j] + ev * xv[j] for j in range(8))
        dvec = dvec + ev * onehot0
        return a, dvec

    SHTL = SLOTS

    def _chunk_rows(ci, b, loff):
        cur_seg0 = bbat[pl.ds(loff, 16)][0]

        def _gate(r):
            xv = [xbuf[b + r, pl.ds(j * 16, 16)] for j in range(8)]
            p = [xv[j] * wv[j] for j in range(8)]
            q = [p[0] + p[1], p[2] + p[3], p[4] + p[5], p[6] + p[7]]
            gv = (q[0] + q[1]) + (q[2] + q[3])
            for ix in bfly:
                gv = gv + gv.at[ix].get(mode="promise_in_bounds")
            return xv, jnp.exp(gv + bv)

        def _group(grp, gcarry):
            cur_seg, k, a, dvec = gcarry
            bat16 = bbat[pl.ds(grp * 16, 16)]
            s_idx = istate[...]
            xv_c, ev_c = _gate(grp * 16)
            for i in range(16):
                if i < 15:
                    xv_n, ev_n = _gate(grp * 16 + i + 1)
                seg = bat16[i]
                new_i = jnp.where(seg != cur_seg, 1, 0)
                newv = lane * 0 + new_i
                cur_seg_v = lane * 0 + cur_seg
                slot = 15 if i == 0 else i - 1
                selv = newv * jnp.where(lane == slot, 1, 0)
                s_idx = jnp.where(selv > 0, cur_seg_v, s_idx)
                k = k + new_i
                keep = jnp.where(new_i == 1, 0.0, 1.0)
                a = tuple(a[j] * keep + ev_c * xv_c[j] for j in range(8))
                dvec = dvec * keep + ev_c * onehot0
                if i < 15:
                    _stage_row(i, a, dvec)
                    xv_c, ev_c = xv_n, ev_n
                cur_seg = seg
            istate[...] = s_idx

            @pl.when(k > 0)
            def _():
                _flush()

            istate[...] = dump_idx
            _stage_row(15, a, dvec)
            return cur_seg, k * 0, a, dvec

        acc0 = tuple(jnp.zeros((16,), jnp.float32) for _ in range(8))
        gcarry = (cur_seg0, jnp.int32(0), acc0, zeros16)
        cur_seg, k, a, dvec = lax.fori_loop(
            loff // 16, R // 16, _group, gcarry)

        _stage_row(0, a, dvec)
        cur_seg_v = lane * 0 + cur_seg
        istate[...] = jnp.where(lane == 0, cur_seg_v, dump_idx)
        _flush()
        istate[...] = dump_idx

    def _ci_row0(c):
        ci = jnp.minimum(wid + c * NW, NCHUNK - 1)
        row0 = jnp.minimum(ci * R, N - R)
        return ci, row0

    def _chunk(c, _):
        ci, row0 = _ci_row0(c)
        loff = (wid + c * NW) * R - row0
        boff = (c % 2) * R
        pltpu.make_async_copy(x_hbm.at[pl.ds(row0, R)],
                              xbuf.at[pl.ds(boff, R)], xsem).wait()
        _, row0n = _ci_row0(c + 1)
        pltpu.async_copy(x_hbm.at[pl.ds(row0n, R)],
                         xbuf.at[pl.ds(R - boff, R)], xsem)
        pltpu.sync_copy(batch_hbm.at[pl.ds(row0, R)], bbat)
        _chunk_rows(ci, boff, loff)
        return 0

    istate[...] = dump_idx
    ntrips = FULL_TRIPS + jnp.where(wid < EXTRA_W, 1, 0)
    _, row0p = _ci_row0(0)
    pltpu.async_copy(x_hbm.at[pl.ds(row0p, R)], xbuf.at[pl.ds(0, R)], xsem)
    lax.fori_loop(0, ntrips, _chunk, 0)
    _, row0d = _ci_row0(ntrips)
    pltpu.make_async_copy(x_hbm.at[pl.ds(row0d, R)],
                          xbuf.at[pl.ds((ntrips % 2) * R, R)], xsem).wait()

    plsc.subcore_barrier()
    out_rows = G // NS
    pltpu.sync_copy(acc_sh.at[pl.ds(sid * out_rows, out_rows)],
                    obuf.at[pl.ds(0, out_rows)])
    pltpu.sync_copy(obuf.at[pl.ds(0, out_rows)],
                    outf_hbm.at[cid, pl.ds(sid * out_rows, out_rows)])
    pltpu.sync_copy(den_sh.at[pl.ds(sid * out_rows, out_rows)],
                    obuf.at[pl.ds(0, out_rows)])
    pltpu.sync_copy(obuf.at[pl.ds(0, out_rows)],
                    outd_hbm.at[cid, pl.ds(sid * out_rows, out_rows)])


@functools.partial(
    pl.kernel,
    out_type=(jax.ShapeDtypeStruct((NC, G, D), jnp.float32),
              jax.ShapeDtypeStruct((NC, G, D), jnp.float32)),
    mesh=plsc.VectorSubcoreMesh(core_axis_name="c", subcore_axis_name="s"),
    scratch_types=[
        pltpu.VMEM((2 * R, D), jnp.float32),
        pltpu.VMEM((R,), jnp.int32),
        pltpu.VMEM((D,), jnp.float32),
        pltpu.VMEM((16,), jnp.float32),
        pltpu.VMEM((SLOTS + 8, D), jnp.float32),
        pltpu.VMEM((SLOTS + 8, D), jnp.float32),
        pltpu.VMEM((SLOTS,), jnp.int32),
        pltpu.VMEM((ACC_ROWS // NS, D), jnp.float32),
        pltpu.VMEM_SHARED((ACC_ROWS, D), jnp.float32),
        pltpu.VMEM_SHARED((ACC_ROWS, D), jnp.float32),
        pltpu.SemaphoreType.DMA,
    ],
)
def _sc_pool(x_hbm, batch_hbm, w_hbm, b_hbm, outf_hbm, outd_hbm, *scratch):
    _sc_body(x_hbm, batch_hbm, w_hbm, b_hbm, outf_hbm, outd_hbm, *scratch)


def _merge_body(f0_ref, f1_ref, d0_ref, d1_ref, o_ref):
    s = f0_ref[...] + f1_ref[...]
    den = d0_ref[...] + d1_ref[...]
    d = jnp.sum(den, axis=1, keepdims=True)
    o_ref[...] = s / (d + 1e-16)


def kernel(x, batch, W_gate, b_gate):
    w128 = W_gate.reshape(D)
    b16 = jnp.broadcast_to(b_gate, (16,)).astype(jnp.float32)
    pf, pd = _sc_pool(x, batch, w128, b16)
    out = pl.pallas_call(
        _merge_body,
        out_shape=jax.ShapeDtypeStruct((G, D), jnp.float32),
    )(pf[0], pf[1], pd[0], pd[1])
    return out


# speedup vs baseline: 12.2102x; 1.0300x over previous
import functools

import jax
import jax.numpy as jnp
from jax import lax
from jax.experimental import pallas as pl
from jax.experimental.pallas import tpu as pltpu
from jax.experimental.pallas import tpu_sc as plsc

N = 100000
D = 128
G = 1024
NC = 2
NS = 16
NW = NC * NS
R = 256
NCHUNK = (N + R - 1) // R
FULL_TRIPS = NCHUNK // NW
EXTRA_W = NCHUNK - FULL_TRIPS * NW
SLOTS = 16
ACC_ROWS = G + 128


def _sc_body(x_hbm, batch_hbm, w_hbm, b_hbm, outf_hbm, outd_hbm,
             xbuf, bbat, wbuf, bbuf, stage, stage_d, istate, obuf,
             acc_sh, den_sh, xsem, fsem):
    cid = lax.axis_index("c")
    sid = lax.axis_index("s")
    wid = cid * NS + sid

    zeros16 = jnp.zeros((16,), jnp.float32)
    rows_per_tec = ACC_ROWS // NS

    def _zrow(r, _):
        for j in range(8):
            obuf[r, pl.ds(j * 16, 16)] = zeros16
        return 0

    lax.fori_loop(0, rows_per_tec, _zrow, 0)
    pltpu.sync_copy(obuf.at[pl.ds(0, rows_per_tec)],
                    acc_sh.at[pl.ds(sid * rows_per_tec, rows_per_tec)])
    pltpu.sync_copy(obuf.at[pl.ds(0, rows_per_tec)],
                    den_sh.at[pl.ds(sid * rows_per_tec, rows_per_tec)])
    for r in range(SLOTS + 8):
        for j in range(8):
            stage_d[r, pl.ds(j * 16, 16)] = zeros16
    plsc.subcore_barrier()

    pltpu.sync_copy(w_hbm, wbuf)
    pltpu.sync_copy(b_hbm, bbuf)
    wv = [wbuf[pl.ds(j * 16, 16)] for j in range(8)]
    bv = bbuf[...]
    lane = lax.broadcasted_iota(jnp.int32, (16,), 0)
    bfly = [lane ^ s for s in (8, 4, 2, 1)]
    onehot0 = jnp.where(lane == 0, 1.0, 0.0)
    dump_idx = lane * 0 + G

    def _flush():
        d1 = pltpu.async_copy(stage.at[pl.ds(0, SLOTS)],
                              acc_sh.at[istate], fsem, add=True)
        d2 = pltpu.async_copy(stage_d.at[pl.ds(0, SLOTS)],
                              den_sh.at[istate], fsem, add=True)
        d1.wait()
        d2.wait()

    def _stage_row(k, a, dvec):
        for j in range(8):
            stage[k, pl.ds(j * 16, 16)] = a[j]
        stage_d[k, pl.ds(0, 16)] = dvec

    def _row_fast(r, b, a, dvec):
        xv = [xbuf[b + r, pl.ds(j * 16, 16)] for j in range(8)]
        gv = xv[0] * wv[0]
        for j in range(1, 8):
            gv = gv + xv[j] * wv[j]
        for ix in bfly:
            gv = gv + gv.at[ix].get(mode="promise_in_bounds")
        ev = jnp.exp(gv + bv)
        a = tuple(a[j] + ev * xv[j] for j in range(8))
        dvec = dvec + ev * onehot0
        return a, dvec

    SHTL = SLOTS

    def _chunk_rows(ci, b, loff):
        cur_seg0 = bbat[pl.ds(loff, 16)][0]

        def _gate(r):
            xv = [xbuf[b + r, pl.ds(j * 16, 16)] for j in range(8)]
            p = [xv[j] * wv[j] for j in range(8)]
            q = [p[0] + p[1], p[2] + p[3], p[4] + p[5], p[6] + p[7]]
            gv = (q[0] + q[1]) + (q[2] + q[3])
            for ix in bfly:
                gv = gv + gv.at[ix].get(mode="promise_in_bounds")
            return xv, jnp.exp(gv + bv)

        def _group(grp, gcarry):
            cur_seg, k, a, dvec = gcarry
            bat16 = bbat[pl.ds(grp * 16, 16)]
            s_idx = istate[...]
            xv_c, ev_c = _gate(grp * 16)
            for i in range(16):
                if i < 15:
                    xv_n, ev_n = _gate(grp * 16 + i + 1)
                seg = bat16[i]
                new_i = jnp.where(seg != cur_seg, 1, 0)
                newv = lane * 0 + new_i
                cur_seg_v = lane * 0 + cur_seg
                slot = 15 if i == 0 else i - 1
                selv = newv * jnp.where(lane == slot, 1, 0)
                s_idx = jnp.where(selv > 0, cur_seg_v, s_idx)
                k = k + new_i
                keep = jnp.where(new_i == 1, 0.0, 1.0)
                a = tuple(a[j] * keep + ev_c * xv_c[j] for j in range(8))
                dvec = dvec * keep + ev_c * onehot0
                if i < 15:
                    _stage_row(i, a, dvec)
                    xv_c, ev_c = xv_n, ev_n
                cur_seg = seg
            istate[...] = s_idx

            @pl.when(k > 0)
            def _():
                _flush()

            istate[...] = dump_idx
            _stage_row(15, a, dvec)
            return cur_seg, k * 0, a, dvec

        acc0 = tuple(jnp.zeros((16,), jnp.float32) for _ in range(8))
        gcarry = (cur_seg0, jnp.int32(0), acc0, zeros16)
        cur_seg, k, a, dvec = lax.fori_loop(
            loff // 16, R // 16, _group, gcarry)

        _stage_row(0, a, dvec)
        cur_seg_v = lane * 0 + cur_seg
        istate[...] = jnp.where(lane == 0, cur_seg_v, dump_idx)
        _flush()
        istate[...] = dump_idx

    def _ci_row0(c):
        ci = jnp.minimum(wid + c * NW, NCHUNK - 1)
        row0 = jnp.minimum(ci * R, N - R)
        return ci, row0

    def _chunk(c, _):
        ci, row0 = _ci_row0(c)
        loff = (wid + c * NW) * R - row0
        boff = (c % 2) * R
        pltpu.make_async_copy(x_hbm.at[pl.ds(row0, R)],
                              xbuf.at[pl.ds(boff, R)], xsem).wait()
        _, row0n = _ci_row0(c + 1)
        pltpu.async_copy(x_hbm.at[pl.ds(row0n, R)],
                         xbuf.at[pl.ds(R - boff, R)], xsem)
        pltpu.sync_copy(batch_hbm.at[pl.ds(row0, R)], bbat)
        _chunk_rows(ci, boff, loff)
        return 0

    istate[...] = dump_idx
    ntrips = FULL_TRIPS + jnp.where(wid < EXTRA_W, 1, 0)
    _, row0p = _ci_row0(0)
    pltpu.async_copy(x_hbm.at[pl.ds(row0p, R)], xbuf.at[pl.ds(0, R)], xsem)
    lax.fori_loop(0, ntrips, _chunk, 0)
    _, row0d = _ci_row0(ntrips)
    pltpu.make_async_copy(x_hbm.at[pl.ds(row0d, R)],
                          xbuf.at[pl.ds((ntrips % 2) * R, R)], xsem).wait()

    plsc.subcore_barrier()
    out_rows = G // NS
    pltpu.sync_copy(acc_sh.at[pl.ds(sid * out_rows, out_rows)],
                    obuf.at[pl.ds(0, out_rows)])
    pltpu.sync_copy(obuf.at[pl.ds(0, out_rows)],
                    outf_hbm.at[cid, pl.ds(sid * out_rows, out_rows)])
    pltpu.sync_copy(den_sh.at[pl.ds(sid * out_rows, out_rows)],
                    obuf.at[pl.ds(0, out_rows)])
    pltpu.sync_copy(obuf.at[pl.ds(0, out_rows)],
                    outd_hbm.at[cid, pl.ds(sid * out_rows, out_rows)])


@functools.partial(
    pl.kernel,
    out_type=(jax.ShapeDtypeStruct((NC, G, D), jnp.float32),
              jax.ShapeDtypeStruct((NC, G, D), jnp.float32)),
    mesh=plsc.VectorSubcoreMesh(core_axis_name="c", subcore_axis_name="s"),
    scratch_types=[
        pltpu.VMEM((2 * R, D), jnp.float32),
        pltpu.VMEM((R,), jnp.int32),
        pltpu.VMEM((D,), jnp.float32),
        pltpu.VMEM((16,), jnp.float32),
        pltpu.VMEM((SLOTS + 8, D), jnp.float32),
        pltpu.VMEM((SLOTS + 8, D), jnp.float32),
        pltpu.VMEM((SLOTS,), jnp.int32),
        pltpu.VMEM((ACC_ROWS // NS, D), jnp.float32),
        pltpu.VMEM_SHARED((ACC_ROWS, D), jnp.float32),
        pltpu.VMEM_SHARED((ACC_ROWS, D), jnp.float32),
        pltpu.SemaphoreType.DMA,
        pltpu.SemaphoreType.DMA,
    ],
)
def _sc_pool(x_hbm, batch_hbm, w_hbm, b_hbm, outf_hbm, outd_hbm, *scratch):
    _sc_body(x_hbm, batch_hbm, w_hbm, b_hbm, outf_hbm, outd_hbm, *scratch)


def _merge_body(f0_ref, f1_ref, d0_ref, d1_ref, o_ref):
    s = f0_ref[...] + f1_ref[...]
    den = d0_ref[...] + d1_ref[...]
    d = jnp.sum(den, axis=1, keepdims=True)
    o_ref[...] = s / (d + 1e-16)


def kernel(x, batch, W_gate, b_gate):
    w128 = W_gate.reshape(D)
    b16 = jnp.broadcast_to(b_gate, (16,)).astype(jnp.float32)
    pf, pd = _sc_pool(x, batch, w128, b16)
    out = pl.pallas_call(
        _merge_body,
        out_shape=jax.ShapeDtypeStruct((G, D), jnp.float32),
    )(pf[0], pf[1], pd[0], pd[1])
    return out


# speedup vs baseline: 12.6977x; 1.0399x over previous
import functools

import jax
import jax.numpy as jnp
from jax import lax
from jax.experimental import pallas as pl
from jax.experimental.pallas import tpu as pltpu
from jax.experimental.pallas import tpu_sc as plsc

N = 100000
D = 128
G = 1024
NC = 2
NS = 16
NW = NC * NS
R = 320
NCHUNK = (N + R - 1) // R
FULL_TRIPS = NCHUNK // NW
EXTRA_W = NCHUNK - FULL_TRIPS * NW
SLOTS = 16
ACC_ROWS = G + 128


def _sc_body(x_hbm, batch_hbm, w_hbm, b_hbm, outf_hbm, outd_hbm,
             xbuf, bbat, wbuf, bbuf, stage, stage_d, istate, obuf,
             acc_sh, den_sh, xsem, fsem):
    cid = lax.axis_index("c")
    sid = lax.axis_index("s")
    wid = cid * NS + sid

    zeros16 = jnp.zeros((16,), jnp.float32)
    rows_per_tec = ACC_ROWS // NS

    def _zrow(r, _):
        for j in range(8):
            obuf[r, pl.ds(j * 16, 16)] = zeros16
        return 0

    lax.fori_loop(0, rows_per_tec, _zrow, 0)
    pltpu.sync_copy(obuf.at[pl.ds(0, rows_per_tec)],
                    acc_sh.at[pl.ds(sid * rows_per_tec, rows_per_tec)])
    pltpu.sync_copy(obuf.at[pl.ds(0, rows_per_tec)],
                    den_sh.at[pl.ds(sid * rows_per_tec, rows_per_tec)])
    for r in range(SLOTS + 8):
        for j in range(8):
            stage_d[r, pl.ds(j * 16, 16)] = zeros16
    plsc.subcore_barrier()

    pltpu.sync_copy(w_hbm, wbuf)
    pltpu.sync_copy(b_hbm, bbuf)
    wv = [wbuf[pl.ds(j * 16, 16)] for j in range(8)]
    bv = bbuf[...]
    lane = lax.broadcasted_iota(jnp.int32, (16,), 0)
    bfly = [lane ^ s for s in (8, 4, 2, 1)]
    onehot0 = jnp.where(lane == 0, 1.0, 0.0)
    dump_idx = lane * 0 + G

    def _flush():
        d1 = pltpu.async_copy(stage.at[pl.ds(0, SLOTS)],
                              acc_sh.at[istate], fsem, add=True)
        d2 = pltpu.async_copy(stage_d.at[pl.ds(0, SLOTS)],
                              den_sh.at[istate], fsem, add=True)
        d1.wait()
        d2.wait()

    def _stage_row(k, a, dvec):
        for j in range(8):
            stage[k, pl.ds(j * 16, 16)] = a[j]
        stage_d[k, pl.ds(0, 16)] = dvec

    def _row_fast(r, b, a, dvec):
        xv = [xbuf[b + r, pl.ds(j * 16, 16)] for j in range(8)]
        gv = xv[0] * wv[0]
        for j in range(1, 8):
            gv = gv + xv[j] * wv[j]
        for ix in bfly:
            gv = gv + gv.at[ix].get(mode="promise_in_bounds")
        ev = jnp.exp(gv + bv)
        a = tuple(a[j] + ev * xv[j] for j in range(8))
        dvec = dvec + ev * onehot0
        return a, dvec

    SHTL = SLOTS

    def _chunk_rows(ci, b, loff):
        cur_seg0 = bbat[pl.ds(loff, 16)][0]

        def _gate(r):
            xv = [xbuf[b + r, pl.ds(j * 16, 16)] for j in range(8)]
            p = [xv[j] * wv[j] for j in range(8)]
            q = [p[0] + p[1], p[2] + p[3], p[4] + p[5], p[6] + p[7]]
            gv = (q[0] + q[1]) + (q[2] + q[3])
            for ix in bfly:
                gv = gv + gv.at[ix].get(mode="promise_in_bounds")
            return xv, jnp.exp(gv + bv)

        def _group(grp, gcarry):
            cur_seg, k, a, dvec = gcarry
            bat16 = bbat[pl.ds(grp * 16, 16)]
            s_idx = istate[...]
            xv_c, ev_c = _gate(grp * 16)
            for i in range(16):
                if i < 15:
                    xv_n, ev_n = _gate(grp * 16 + i + 1)
                seg = bat16[i]
                new_i = jnp.where(seg != cur_seg, 1, 0)
                newv = lane * 0 + new_i
                cur_seg_v = lane * 0 + cur_seg
                slot = 15 if i == 0 else i - 1
                selv = newv * jnp.where(lane == slot, 1, 0)
                s_idx = jnp.where(selv > 0, cur_seg_v, s_idx)
                k = k + new_i
                keep = jnp.where(new_i == 1, 0.0, 1.0)
                a = tuple(a[j] * keep + ev_c * xv_c[j] for j in range(8))
                dvec = dvec * keep + ev_c * onehot0
                if i < 15:
                    _stage_row(i, a, dvec)
                    xv_c, ev_c = xv_n, ev_n
                cur_seg = seg
            istate[...] = s_idx

            @pl.when(k > 0)
            def _():
                _flush()

            istate[...] = dump_idx
            _stage_row(15, a, dvec)
            return cur_seg, k * 0, a, dvec

        acc0 = tuple(jnp.zeros((16,), jnp.float32) for _ in range(8))
        gcarry = (cur_seg0, jnp.int32(0), acc0, zeros16)
        cur_seg, k, a, dvec = lax.fori_loop(
            loff // 16, R // 16, _group, gcarry)

        _stage_row(0, a, dvec)
        cur_seg_v = lane * 0 + cur_seg
        istate[...] = jnp.where(lane == 0, cur_seg_v, dump_idx)
        _flush()
        istate[...] = dump_idx

    def _ci_row0(c):
        ci = jnp.minimum(wid + c * NW, NCHUNK - 1)
        row0 = jnp.minimum(ci * R, N - R)
        return ci, row0

    def _chunk(c, _):
        ci, row0 = _ci_row0(c)
        loff = (wid + c * NW) * R - row0
        boff = (c % 2) * R
        pltpu.make_async_copy(x_hbm.at[pl.ds(row0, R)],
                              xbuf.at[pl.ds(boff, R)], xsem).wait()
        _, row0n = _ci_row0(c + 1)
        pltpu.async_copy(x_hbm.at[pl.ds(row0n, R)],
                         xbuf.at[pl.ds(R - boff, R)], xsem)
        pltpu.sync_copy(batch_hbm.at[pl.ds(row0, R)], bbat)
        _chunk_rows(ci, boff, loff)
        return 0

    istate[...] = dump_idx
    ntrips = FULL_TRIPS + jnp.where(wid < EXTRA_W, 1, 0)
    _, row0p = _ci_row0(0)
    pltpu.async_copy(x_hbm.at[pl.ds(row0p, R)], xbuf.at[pl.ds(0, R)], xsem)
    lax.fori_loop(0, ntrips, _chunk, 0)
    _, row0d = _ci_row0(ntrips)
    pltpu.make_async_copy(x_hbm.at[pl.ds(row0d, R)],
                          xbuf.at[pl.ds((ntrips % 2) * R, R)], xsem).wait()

    plsc.subcore_barrier()
    out_rows = G // NS
    pltpu.sync_copy(acc_sh.at[pl.ds(sid * out_rows, out_rows)],
                    obuf.at[pl.ds(0, out_rows)])
    pltpu.sync_copy(obuf.at[pl.ds(0, out_rows)],
                    outf_hbm.at[cid, pl.ds(sid * out_rows, out_rows)])
    pltpu.sync_copy(den_sh.at[pl.ds(sid * out_rows, out_rows)],
                    obuf.at[pl.ds(0, out_rows)])
    pltpu.sync_copy(obuf.at[pl.ds(0, out_rows)],
                    outd_hbm.at[cid, pl.ds(sid * out_rows, out_rows)])


@functools.partial(
    pl.kernel,
    out_type=(jax.ShapeDtypeStruct((NC, G, D), jnp.float32),
              jax.ShapeDtypeStruct((NC, G, D), jnp.float32)),
    mesh=plsc.VectorSubcoreMesh(core_axis_name="c", subcore_axis_name="s"),
    scratch_types=[
        pltpu.VMEM((2 * R, D), jnp.float32),
        pltpu.VMEM((R,), jnp.int32),
        pltpu.VMEM((D,), jnp.float32),
        pltpu.VMEM((16,), jnp.float32),
        pltpu.VMEM((SLOTS + 8, D), jnp.float32),
        pltpu.VMEM((SLOTS + 8, D), jnp.float32),
        pltpu.VMEM((SLOTS,), jnp.int32),
        pltpu.VMEM((ACC_ROWS // NS, D), jnp.float32),
        pltpu.VMEM_SHARED((ACC_ROWS, D), jnp.float32),
        pltpu.VMEM_SHARED((ACC_ROWS, D), jnp.float32),
        pltpu.SemaphoreType.DMA,
        pltpu.SemaphoreType.DMA,
    ],
)
def _sc_pool(x_hbm, batch_hbm, w_hbm, b_hbm, outf_hbm, outd_hbm, *scratch):
    _sc_body(x_hbm, batch_hbm, w_hbm, b_hbm, outf_hbm, outd_hbm, *scratch)


def _merge_body(f0_ref, f1_ref, d0_ref, d1_ref, o_ref):
    s = f0_ref[...] + f1_ref[...]
    den = d0_ref[...] + d1_ref[...]
    d = jnp.sum(den, axis=1, keepdims=True)
    o_ref[...] = s / (d + 1e-16)


def kernel(x, batch, W_gate, b_gate):
    w128 = W_gate.reshape(D)
    b16 = jnp.broadcast_to(b_gate, (16,)).astype(jnp.float32)
    pf, pd = _sc_pool(x, batch, w128, b16)
    out = pl.pallas_call(
        _merge_body,
        out_shape=jax.ShapeDtypeStruct((G, D), jnp.float32),
    )(pf[0], pf[1], pd[0], pd[1])
    return out
